# baseline (speedup 1.0000x reference)
import jax
import jax.numpy as jnp
from jax.experimental import pallas as pl

_DECAY = 1.0


def _infectivity_kernel(ti_ref, tjs_ref, cjs_ref, emb_ref, out_ref):
    gt = jnp.exp(_DECAY * (tjs_ref[:] - ti_ref[:]))
    hist = cjs_ref[0].astype(jnp.float32)
    phi_c = jax.lax.dot_general(
        hist, emb_ref[:], (((1,), (1,)), ((), ())),
        preferred_element_type=jnp.float32)
    out_ref[:] = jax.lax.dot_general(
        phi_c, gt, (((0,), (1,)), ((), ())),
        preferred_element_type=jnp.float32)


def kernel(ti, tjs, ci, cjs, emb_weight):
    B = ti.shape[0]
    N = emb_weight.shape[0]
    out2d = pl.pallas_call(
        _infectivity_kernel,
        out_shape=jax.ShapeDtypeStruct((N, B), jnp.float32),
    )(ti, tjs, cjs, emb_weight)
    return out2d[:, :, None]
